# baseline (speedup 1.0000x reference)
import jax
import jax.numpy as jnp
from jax import lax
from jax.experimental import pallas as pl

_R = 256
_W = 384
_C = 128


def _mi_tc_body(x_ref, mk_ref, out_first, out_ye):
    X = x_ref[...]
    col = lax.iota(jnp.int32, _W)
    tgt = col * 3 - jnp.int32(383) * (col >> 7)
    row = lax.broadcasted_iota(jnp.int32, (_W, _W), 0)
    S = jnp.where(row == tgt[None, :], jnp.float32(1.0), jnp.float32(0.0))
    Y = jnp.dot(X, S, preferred_element_type=jnp.float32)
    a0 = Y[:, 0:_C]
    a1 = Y[:, _C:2 * _C]
    a2 = Y[:, 2 * _C:3 * _C]
    mf = jnp.where(mk_ref[...] != 0, jnp.float32(1.0), jnp.float32(0.0))
    mx = jnp.maximum(a0, jnp.maximum(a1, a2))
    e0 = jnp.exp(a0 - mx)
    e1 = jnp.exp(a1 - mx)
    e2 = jnp.exp(a2 - mx)
    sinv = jnp.float32(1.0) / (e0 + e1 + e2)
    p0 = e0 * sinv
    p12 = (e1 + e2) * sinv
    h = -(p0 * jnp.log(p0) + p12 * jnp.log(p12))
    count = jnp.sum(mf)
    cinv = jnp.float32(1.0) / count
    condi = jnp.sum(h * mf) * cinv
    y0 = jnp.sum(p0 * mf) * cinv
    y1 = jnp.sum(p12 * mf) * cinv
    ye = -(y0 * jnp.log(y0) + y1 * jnp.log(y1))
    first = jnp.where(ye < jnp.float32(0.5), condi - ye, condi)
    out_first[...] = jnp.broadcast_to(first, (1, 1))
    out_ye[...] = jnp.broadcast_to(ye, (1, 1))


def kernel(logits, masks):
    X = logits.reshape(_R, _W)
    mk = masks.reshape(_R, _C).astype(jnp.int32)
    first, ye = pl.pallas_call(
        _mi_tc_body,
        out_shape=(jax.ShapeDtypeStruct((1, 1), jnp.float32),
                   jax.ShapeDtypeStruct((1, 1), jnp.float32)),
    )(X, mk)
    return (first[0, 0], ye[0, 0])


# speedup vs baseline: 9.1697x; 9.1697x over previous
import jax
import jax.numpy as jnp
from jax.experimental import pallas as pl


def _mi_tc_body(lt_ref, mk_ref, out_first, out_ye):
    a0 = lt_ref[0]
    a1 = lt_ref[1]
    a2 = lt_ref[2]
    mf = jnp.where(mk_ref[...] != 0, jnp.float32(1.0), jnp.float32(0.0))
    mx = jnp.maximum(a0, jnp.maximum(a1, a2))
    e0 = jnp.exp(a0 - mx)
    e1 = jnp.exp(a1 - mx)
    e2 = jnp.exp(a2 - mx)
    sinv = jnp.float32(1.0) / (e0 + e1 + e2)
    p0 = e0 * sinv
    p12 = (e1 + e2) * sinv
    h = -(p0 * jnp.log(p0) + p12 * jnp.log(p12))
    count = jnp.sum(mf)
    cinv = jnp.float32(1.0) / count
    condi = jnp.sum(h * mf) * cinv
    y0 = jnp.sum(p0 * mf) * cinv
    y1 = jnp.sum(p12 * mf) * cinv
    ye = -(y0 * jnp.log(y0) + y1 * jnp.log(y1))
    first = jnp.where(ye < jnp.float32(0.5), condi - ye, condi)
    out_first[...] = jnp.broadcast_to(first, (1, 1))
    out_ye[...] = jnp.broadcast_to(ye, (1, 1))


def kernel(logits, masks):
    lt = logits.transpose(2, 0, 1)
    mk = masks.astype(jnp.int32)
    first, ye = pl.pallas_call(
        _mi_tc_body,
        out_shape=(jax.ShapeDtypeStruct((1, 1), jnp.float32),
                   jax.ShapeDtypeStruct((1, 1), jnp.float32)),
    )(lt, mk)
    return (first[0, 0], ye[0, 0])
